# baseline (speedup 1.0000x reference)
import functools

import jax
import jax.numpy as jnp
from jax import lax
from jax.experimental import pallas as pl
from jax.experimental.pallas import tpu as pltpu
from jax.experimental.pallas import tpu_sc as plsc

B, F, D = 4096, 26, 32
NC, NS = 2, 16
NW = NC * NS
RPW = B // NW
NG = RPW // 16


def _fm_body(sparse_hbm, embed_hbm, w_hbm, out_hbm,
             slab_v, idx_v, gath_v, emb_v, out_v, sem):
    wid = lax.axis_index("s") * NC + lax.axis_index("c")
    base = wid * RPW
    pltpu.sync_copy(sparse_hbm.at[pl.ds(base * F, RPW * F)], slab_v)
    pltpu.sync_copy(embed_hbm.at[pl.ds(base * F * D, RPW * F * D)], emb_v)

    iota = lax.iota(jnp.int32, 16)

    for g in range(NG):
        row0 = (g * 16 + iota) * F
        for f in range(F):
            idx_v[f, pl.ds(g * 16, 16)] = plsc.load_gather(slab_v, [row0 + f])

    copies = [
        pltpu.async_copy(w_hbm.at[idx_v.at[f]], gath_v.at[f], sem)
        for f in range(F)
    ]

    for g in range(NG):
        rbase = (g * 16 + iota) * (F * D)

        def dbody(d, acc2, rbase=rbase):
            s = jnp.zeros((16,), jnp.float32)
            q = jnp.zeros((16,), jnp.float32)
            for f in range(F):
                v = plsc.load_gather(emb_v, [rbase + (f * D + d)])
                s = s + v
                q = q + v * v
            return acc2 + (s * s - q)

        acc2 = lax.fori_loop(0, D, dbody, jnp.zeros((16,), jnp.float32))
        out_v[pl.ds(g * 16, 16)] = 0.5 * acc2

    for c in copies:
        c.wait()

    for g in range(NG):
        fo = jnp.zeros((16,), jnp.float32)
        for f in range(F):
            fo = fo + gath_v[f, pl.ds(g * 16, 16)]
        out_v[pl.ds(g * 16, 16)] = out_v[pl.ds(g * 16, 16)] + fo

    pltpu.sync_copy(out_v, out_hbm.at[pl.ds(base, RPW)])


@jax.jit
def kernel(sparse_inputs, embed_inputs, w):
    run = pl.kernel(
        _fm_body,
        out_type=jax.ShapeDtypeStruct((B,), jnp.float32),
        mesh=plsc.VectorSubcoreMesh(core_axis_name="c", subcore_axis_name="s"),
        scratch_types=[
            pltpu.VMEM((RPW * F,), jnp.int32),
            pltpu.VMEM((F, RPW), jnp.int32),
            pltpu.VMEM((F, RPW), jnp.float32),
            pltpu.VMEM((RPW * F * D,), jnp.float32),
            pltpu.VMEM((RPW,), jnp.float32),
            pltpu.SemaphoreType.DMA,
        ],
        compiler_params=pltpu.CompilerParams(needs_layout_passes=False),
    )
    out = run(sparse_inputs.reshape(-1), embed_inputs.reshape(-1),
              w.reshape(-1))
    return out.reshape(B, 1)


# speedup vs baseline: 2.2748x; 2.2748x over previous
import functools

import jax
import jax.numpy as jnp
from jax import lax
from jax.experimental import pallas as pl
from jax.experimental.pallas import tpu as pltpu
from jax.experimental.pallas import tpu_sc as plsc

B, F, D = 4096, 26, 32
NC, NS = 2, 16
NW = NC * NS
RPW = B // NW
NL = RPW // 16


def _fm_body(st_hbm, et_hbm, w_hbm, out_hbm,
             slab_v, gath_v, emb_v, out_v, sem):
    wid = lax.axis_index("s") * NC + lax.axis_index("c")
    base = wid * RPW
    pltpu.sync_copy(st_hbm.at[:, pl.ds(base, RPW)], slab_v)
    pltpu.sync_copy(et_hbm.at[:, :, pl.ds(base, RPW)], emb_v)

    copies = [
        pltpu.async_copy(w_hbm.at[slab_v.at[f]], gath_v.at[f], sem)
        for f in range(F)
    ]

    zero16 = jnp.zeros((16,), jnp.float32)
    for l in range(NL):
        def dbody(d, acc2, l=l):
            s = zero16
            q = zero16
            for f in range(F):
                v = emb_v[f, d, pl.ds(l * 16, 16)]
                s = s + v
                q = q + v * v
            return acc2 + (s * s - q)

        acc2 = lax.fori_loop(0, D, dbody, zero16)
        out_v[pl.ds(l * 16, 16)] = 0.5 * acc2

    for c in copies:
        c.wait()

    for l in range(NL):
        fo = zero16
        for f in range(F):
            fo = fo + gath_v[f, pl.ds(l * 16, 16)]
        out_v[pl.ds(l * 16, 16)] = out_v[pl.ds(l * 16, 16)] + fo

    pltpu.sync_copy(out_v, out_hbm.at[pl.ds(base, RPW)])


@jax.jit
def kernel(sparse_inputs, embed_inputs, w):
    run = pl.kernel(
        _fm_body,
        out_type=jax.ShapeDtypeStruct((B,), jnp.float32),
        mesh=plsc.VectorSubcoreMesh(core_axis_name="c", subcore_axis_name="s"),
        scratch_types=[
            pltpu.VMEM((F, RPW), jnp.int32),
            pltpu.VMEM((F, RPW), jnp.float32),
            pltpu.VMEM((F, D, RPW), jnp.float32),
            pltpu.VMEM((RPW,), jnp.float32),
            pltpu.SemaphoreType.DMA,
        ],
        compiler_params=pltpu.CompilerParams(needs_layout_passes=False),
    )
    st = sparse_inputs.T
    et = embed_inputs.transpose(1, 2, 0)
    wf = w[:, 0]
    return run(st, et, wf).reshape(B, 1)


# speedup vs baseline: 5.0034x; 2.1995x over previous
import functools

import jax
import jax.numpy as jnp
from jax import lax
from jax.experimental import pallas as pl
from jax.experimental.pallas import tpu as pltpu
from jax.experimental.pallas import tpu_sc as plsc

B, F, D = 4096, 26, 32
NC, NS = 2, 16
NW = NC * NS
RPW = B // NW
NL = RPW // 16


def _fm_body(st_hbm, et_hbm, w_hbm, out_hbm,
             slab_v, gath_v, emb_v, out_v, sem):
    wid = lax.axis_index("s") * NC + lax.axis_index("c")
    base = wid * RPW
    pltpu.sync_copy(st_hbm.at[:, pl.ds(base, RPW)], slab_v)
    pltpu.sync_copy(et_hbm.at[:, :, pl.ds(base, RPW)], emb_v)

    copies = [
        pltpu.async_copy(w_hbm.at[0].at[slab_v.at[f]], gath_v.at[f], sem)
        for f in range(F)
    ]

    zero16 = jnp.zeros((16,), jnp.float32)
    for l in range(NL):
        def dbody(d, acc2, l=l):
            s = zero16
            q = zero16
            for f in range(F):
                v = emb_v[f, d, pl.ds(l * 16, 16)]
                s = s + v
                q = q + v * v
            return acc2 + (s * s - q)

        acc2 = lax.fori_loop(0, D, dbody, zero16)
        out_v[pl.ds(l * 16, 16)] = 0.5 * acc2

    for c in copies:
        c.wait()

    for l in range(NL):
        fo = zero16
        for f in range(F):
            fo = fo + gath_v[f, pl.ds(l * 16, 16)]
        out_v[pl.ds(l * 16, 16)] = out_v[pl.ds(l * 16, 16)] + fo

    pltpu.sync_copy(out_v, out_hbm.at[pl.ds(base, RPW)])


@jax.jit
def kernel(sparse_inputs, embed_inputs, w):
    run = pl.kernel(
        _fm_body,
        out_type=jax.ShapeDtypeStruct((B,), jnp.float32),
        mesh=plsc.VectorSubcoreMesh(core_axis_name="c", subcore_axis_name="s"),
        scratch_types=[
            pltpu.VMEM((F, RPW), jnp.int32),
            pltpu.VMEM((F, RPW), jnp.float32),
            pltpu.VMEM((F, D, RPW), jnp.float32),
            pltpu.VMEM((RPW,), jnp.float32),
            pltpu.SemaphoreType.DMA,
        ],
        compiler_params=pltpu.CompilerParams(needs_layout_passes=False),
    )
    st = sparse_inputs.T
    et = embed_inputs.transpose(1, 2, 0)
    wt = w.T
    return run(st, et, wt).reshape(B, 1)
